# baseline (speedup 1.0000x reference)
import functools

import jax
import jax.numpy as jnp
from jax import lax
from jax.experimental import pallas as pl
from jax.experimental.pallas import tpu as pltpu
from jax.experimental.pallas import tpu_sc as plsc

N = 100000
E = 1600000
NUM_GRAPHS = 256
H = 32

NC, NS = 2, 16
NW = NC * NS

NP = 102400
EP = 1638400
TROWS = EP // NW // 128
GRP = 8
NGRP = TROWS // GRP
PSLOTS = 2560
BLK = 512
NBLK = NP // BLK

_f32 = jnp.float32


@functools.cache
def _get_sc_deg():
    mesh = plsc.VectorSubcoreMesh(core_axis_name="c", subcore_axis_name="s")
    return functools.partial(
        pl.kernel,
        out_type=jax.ShapeDtypeStruct((NC, NP), _f32),
        mesh=mesh,
        compiler_params=pltpu.CompilerParams(use_tc_tiling_on_sc=False),
        scratch_types=[
            pltpu.VMEM_SHARED((NP,), _f32),
            pltpu.VMEM((128,), _f32),
            pltpu.VMEM((128,), _f32)]
            + [pltpu.VMEM((128,), jnp.int32) for _ in range(2 * GRP)]
            + [pltpu.SemaphoreType.DMA, pltpu.SemaphoreType.DMA],
    )(_sc_deg_body)


def _sc_deg_body(dstf, zer, one, degp, accd, zb, ob, *rest):
    djs = (rest[:GRP], rest[GRP:2 * GRP])
    semi, sems = rest[2 * GRP], rest[2 * GRP + 1]
    c = lax.axis_index("c")
    s = lax.axis_index("s")
    wid = c * NS + s
    pltpu.sync_copy(zer, zb)
    pltpu.sync_copy(one, ob)
    zbase = s * (NP // NS)
    zhs = [
        pltpu.async_copy(zb, accd.at[pl.ds(zbase + i * 128, 128)], semi)
        for i in range(NP // NS // 128)
    ]
    for h in zhs:
        h.wait()
    plsc.subcore_barrier()

    eb0 = wid * TROWS * 128

    def fire_idx(k, g):
        eb = eb0 + g * (GRP * 128)
        for j in range(GRP):
            pltpu.async_copy(dstf.at[pl.ds(eb + j * 128, 128)], djs[k][j], semi)

    def wait_idx(k):
        for j in range(GRP):
            pltpu.make_async_copy(dstf.at[pl.ds(0, 128)], djs[k][j], semi).wait()

    def fire_scat(k):
        for j in range(GRP):
            pltpu.async_copy(ob, accd.at[djs[k][j]], sems, add=True)

    def drain_scat(k):
        for j in range(GRP):
            pltpu.make_async_copy(ob, accd.at[djs[k][j]], sems).wait()

    def phase(k, g, drain):
        wait_idx(k)
        if drain:
            drain_scat(1 - k)
        fire_idx(1 - k, lax.rem(g + 1, NGRP))
        fire_scat(k)

    fire_idx(0, 0)
    phase(0, 0, False)

    @pl.loop(0, (NGRP - 2) // 2)
    def _g(i):
        phase(1, 1 + 2 * i, True)
        phase(0, 2 + 2 * i, True)

    phase(1, NGRP - 1, True)
    drain_scat(1)
    wait_idx(0)
    plsc.subcore_barrier()
    pltpu.sync_copy(accd.at[pl.ds(zbase, NP // NS)], degp.at[c, pl.ds(zbase, NP // NS)])


@functools.cache
def _make_edge_pass():
    D = 8
    mesh = plsc.VectorSubcoreMesh(core_axis_name="c", subcore_axis_name="s")

    @functools.partial(
        pl.kernel,
        out_type=jax.ShapeDtypeStruct((NC, NP, D), _f32),
        mesh=mesh,
        compiler_params=pltpu.CompilerParams(use_tc_tiling_on_sc=False),
        scratch_types=[
            pltpu.VMEM_SHARED((NP, D), _f32),
            pltpu.VMEM((128, D), _f32)]
            + [pltpu.VMEM((128,), jnp.int32) for _ in range(4 * GRP)]
            + [pltpu.VMEM((128, D), _f32) for _ in range(2 * GRP)]
            + [pltpu.SemaphoreType.DMA for _ in range(3)],
    )
    def _edge(srcf, dstf, tab, zer, out, acc, zb, *rest):
        sjs = (rest[:GRP], rest[GRP:2 * GRP])
        djs = (rest[2 * GRP:3 * GRP], rest[3 * GRP:4 * GRP])
        rows = (rest[4 * GRP:5 * GRP], rest[5 * GRP:6 * GRP])
        semi, semg, sems = rest[6 * GRP:6 * GRP + 3]
        c = lax.axis_index("c")
        s = lax.axis_index("s")
        wid = c * NS + s
        pltpu.sync_copy(zer, zb)
        zbase = s * (NP // NS)
        zhs = [
            pltpu.async_copy(zb, acc.at[pl.ds(zbase + i * 128, 128)], semi)
            for i in range(NP // NS // 128)
        ]
        for h in zhs:
            h.wait()
        plsc.subcore_barrier()

        eb0 = wid * TROWS * 128

        def fire_idx(k, g):
            eb = eb0 + g * (GRP * 128)
            for j in range(GRP):
                pltpu.async_copy(srcf.at[pl.ds(eb + j * 128, 128)], sjs[k][j], semi)
                pltpu.async_copy(dstf.at[pl.ds(eb + j * 128, 128)], djs[k][j], semi)

        def wait_idx(k):
            for j in range(GRP):
                pltpu.make_async_copy(srcf.at[pl.ds(0, 128)], sjs[k][j], semi).wait()
                pltpu.make_async_copy(dstf.at[pl.ds(0, 128)], djs[k][j], semi).wait()

        def drain_scat(k):
            for j in range(GRP):
                pltpu.make_async_copy(rows[k][j], acc.at[djs[k][j]], sems).wait()

        def phase(k, g, drain):
            wait_idx(k)
            ghs = [
                pltpu.async_copy(tab.at[sjs[k][j]], rows[k][j], semg)
                for j in range(GRP)
            ]
            if drain:
                drain_scat(1 - k)
            fire_idx(1 - k, lax.rem(g + 1, NGRP))
            for h in ghs:
                h.wait()
            for j in range(GRP):
                pltpu.async_copy(rows[k][j], acc.at[djs[k][j]], sems, add=True)

        fire_idx(0, 0)
        phase(0, 0, False)

        @pl.loop(0, (NGRP - 2) // 2)
        def _g(i):
            phase(1, 1 + 2 * i, True)
            phase(0, 2 + 2 * i, True)

        phase(1, NGRP - 1, True)
        drain_scat(1)
        wait_idx(0)
        plsc.subcore_barrier()
        pltpu.sync_copy(
            acc.at[pl.ds(zbase, NP // NS)], out.at[c, pl.ds(zbase, NP // NS)]
        )

    return _edge


@functools.cache
def _get_sc_pool():
    mesh = plsc.VectorSubcoreMesh(core_axis_name="c", subcore_axis_name="s")
    return functools.partial(
        pl.kernel,
        out_type=(
            jax.ShapeDtypeStruct((NC, PSLOTS), _f32),
            jax.ShapeDtypeStruct((NC, PSLOTS), _f32),
        ),
        mesh=mesh,
        compiler_params=pltpu.CompilerParams(use_tc_tiling_on_sc=False),
        scratch_types=[
            pltpu.VMEM_SHARED((PSLOTS,), _f32),
            pltpu.VMEM_SHARED((PSLOTS,), _f32),
            pltpu.VMEM((128,), _f32),
            pltpu.VMEM((128,), _f32),
            pltpu.VMEM((1024,), _f32),
            pltpu.VMEM((1024,), _f32),
            pltpu.VMEM((1024,), _f32),
            pltpu.VMEM((1024,), _f32)]
            + [pltpu.VMEM((128,), _f32) for _ in range(GRP)]
            + [pltpu.VMEM((128,), jnp.int32) for _ in range(GRP)]
            + [pltpu.SemaphoreType.DMA, pltpu.SemaphoreType.DMA],
    )(_sc_pool_body)


def _sc_pool_body(s2a, s2b, d8, w8, bidxf, zer, one, poolp, cntp,
                  accp, accq, zb, ob, ab, bb, db, wb, *rest):
    lbs = rest[:GRP]
    ibs = rest[GRP:2 * GRP]
    semi, sems = rest[2 * GRP], rest[2 * GRP + 1]
    c = lax.axis_index("c")
    s = lax.axis_index("s")
    wid = c * NS + s
    pltpu.sync_copy(zer, zb)
    pltpu.sync_copy(one, ob)

    @pl.when(s == 0)
    def _zp():
        for k in range(PSLOTS // 128):
            pltpu.sync_copy(zb, accp.at[pl.ds(k * 128, 128)])

    @pl.when(s == 1)
    def _zq():
        for k in range(PSLOTS // 128):
            pltpu.sync_copy(zb, accq.at[pl.ds(k * 128, 128)])

    plsc.subcore_barrier()

    eb0 = wid * (NP * 8 // NW)

    def fire_data(g):
        eb = eb0 + g * 1024
        pltpu.async_copy(s2a.at[pl.ds(eb, 1024)], ab, semi)
        pltpu.async_copy(s2b.at[pl.ds(eb, 1024)], bb, semi)
        pltpu.async_copy(d8.at[pl.ds(eb, 1024)], db, semi)
        pltpu.async_copy(w8.at[pl.ds(eb, 1024)], wb, semi)

    def fire_idx(g):
        eb = eb0 + g * 1024
        for j in range(GRP):
            pltpu.async_copy(bidxf.at[pl.ds(eb + j * 128, 128)], ibs[j], semi)

    def wait_loads():
        for r in (ab, bb, db, wb):
            pltpu.make_async_copy(s2a.at[pl.ds(0, 1024)], r, semi).wait()
        for j in range(GRP):
            pltpu.make_async_copy(bidxf.at[pl.ds(0, 128)], ibs[j], semi).wait()

    def drain_scat():
        for j in range(GRP):
            pltpu.make_async_copy(lbs[j], accp.at[ibs[j]], sems).wait()
            pltpu.make_async_copy(ob, accq.at[ibs[j]], sems).wait()

    def body(g, drain):
        fire_data(g)
        if drain:
            drain_scat()
        fire_idx(g)
        wait_loads()
        for j in range(GRP):
            for k in range(8):
                sl = pl.ds(j * 128 + k * 16, 16)
                lbs[j][pl.ds(k * 16, 16)] = db[sl] * (ab[sl] + bb[sl]) + wb[sl]
        for j in range(GRP):
            pltpu.async_copy(lbs[j], accp.at[ibs[j]], sems, add=True)
            pltpu.async_copy(ob, accq.at[ibs[j]], sems, add=True)

    body(0, False)

    @pl.loop(1, NP * 8 // NW // 1024)
    def _p(i):
        body(i, True)

    drain_scat()
    plsc.subcore_barrier()

    @pl.when(s == 0)
    def _op():
        pltpu.sync_copy(accp, poolp.at[c])

    @pl.when(s == 1)
    def _oq():
        pltpu.sync_copy(accq, cntp.at[c])


def _tc1_body(a_ref, p_ref, degp_ref, xs_ref, dinv_ref):
    i = pl.program_id(0)
    deg = degp_ref[0] + degp_ref[1] + 1.0
    dinv = lax.rsqrt(deg)
    x = jnp.concatenate(
        [a_ref[...], p_ref[...], jnp.zeros((BLK, 4), _f32)], axis=-1
    )
    rid = lax.broadcasted_iota(jnp.int32, (BLK, 1), 0) + i * BLK
    xs_ref[...] = jnp.where(rid < N, x * dinv, 0.0)
    dinv_ref[...] = dinv


def _tc1(a2, pos2, degp3):
    return pl.pallas_call(
        _tc1_body,
        grid=(NBLK,),
        in_specs=[
            pl.BlockSpec((BLK, 1), lambda i: (i, 0)),
            pl.BlockSpec((BLK, 3), lambda i: (i, 0)),
            pl.BlockSpec((2, BLK, 1), lambda i: (0, i, 0)),
        ],
        out_specs=[
            pl.BlockSpec((BLK, 8), lambda i: (i, 0)),
            pl.BlockSpec((BLK, 1), lambda i: (i, 0)),
        ],
        out_shape=[
            jax.ShapeDtypeStruct((NP, 8), _f32),
            jax.ShapeDtypeStruct((NP, 1), _f32),
        ],
    )(a2, pos2, degp3)


def _tc2_body(s1_ref, xs_ref, dinv_ref, bat_ref, W1_ref, b1_ref, W2_ref,
              Wl_ref, b2_ref, bl_ref, z_ref, w_ref, d8_ref, bidx_ref):
    dinv = dinv_ref[...]
    agg1 = (s1_ref[0, :, :4] + s1_ref[1, :, :4] + xs_ref[:, :4]) * dinv
    W1 = W1_ref[...]
    x1 = b1_ref[...]
    for k in range(4):
        x1 = x1 + agg1[:, k:k + 1] * W1[k:k + 1, :]
    y = jnp.maximum(x1, 0.0)
    M = jnp.dot(W2_ref[...], Wl_ref[...], preferred_element_type=_f32)
    z = jnp.dot(y * dinv, M, preferred_element_type=_f32)
    cvec = jnp.dot(b2_ref[...], Wl_ref[...], preferred_element_type=_f32) + bl_ref[...]
    z_ref[...] = z
    w_ref[...] = dinv * z + cvec
    d8_ref[...] = jnp.broadcast_to(dinv, (BLK, 8))
    bidx_ref[...] = bat_ref[...] * 8 + lax.broadcasted_iota(jnp.int32, (BLK, 8), 1)


def _tc2(s1p, xs, dinv, bat2, W1, b1, W2, Wl, b2, bl):
    wspec = lambda r, c_: pl.BlockSpec((r, c_), lambda i: (0, 0))
    return pl.pallas_call(
        _tc2_body,
        grid=(NBLK,),
        in_specs=[
            pl.BlockSpec((2, BLK, 8), lambda i: (0, i, 0)),
            pl.BlockSpec((BLK, 8), lambda i: (i, 0)),
            pl.BlockSpec((BLK, 1), lambda i: (i, 0)),
            pl.BlockSpec((BLK, 1), lambda i: (i, 0)),
            wspec(4, 32), wspec(1, 32), wspec(32, 32),
            wspec(32, 8), wspec(1, 32), wspec(1, 8),
        ],
        out_specs=[
            pl.BlockSpec((BLK, 8), lambda i: (i, 0)),
            pl.BlockSpec((BLK, 8), lambda i: (i, 0)),
            pl.BlockSpec((BLK, 8), lambda i: (i, 0)),
            pl.BlockSpec((BLK, 8), lambda i: (i, 0)),
        ],
        out_shape=[
            jax.ShapeDtypeStruct((NP, 8), _f32),
            jax.ShapeDtypeStruct((NP, 8), _f32),
            jax.ShapeDtypeStruct((NP, 8), _f32),
            jax.ShapeDtypeStruct((NP, 8), jnp.int32),
        ],
    )(s1p, xs, dinv, bat2, W1, b1, W2, Wl, b2, bl)


def _tc3_body(pool_ref, cnt_ref, out_ref):
    sums = pool_ref[0] + pool_ref[1]
    cnt = (cnt_ref[0] + cnt_ref[1])[:, 0:1]
    pooled = sums[:NUM_GRAPHS] / jnp.maximum(cnt[:NUM_GRAPHS], 1.0)
    m = jnp.max(pooled, axis=1, keepdims=True)
    lse = jnp.log(jnp.sum(jnp.exp(pooled - m), axis=1, keepdims=True)) + m
    out_ref[...] = pooled - lse


def _tc3(poolp3, cntp3):
    return pl.pallas_call(
        _tc3_body,
        grid=(1,),
        in_specs=[
            pl.BlockSpec((2, PSLOTS // 8, 8), lambda i: (0, 0, 0)),
            pl.BlockSpec((2, PSLOTS // 8, 8), lambda i: (0, 0, 0)),
        ],
        out_specs=pl.BlockSpec((NUM_GRAPHS, 8), lambda i: (0, 0)),
        out_shape=jax.ShapeDtypeStruct((NUM_GRAPHS, 8), _f32),
    )(poolp3, cntp3)


def kernel(atomic_numbers, pos, edge_index, batch, W1, b1, W2, b2, Wl, bl):
    src = edge_index[0]
    dst = edge_index[1]
    pad_e = EP - E
    srcf = jnp.concatenate([src, jnp.zeros((pad_e,), jnp.int32)])
    dstf = jnp.concatenate([dst, jnp.full((pad_e,), N, jnp.int32)])
    bat_np = jnp.concatenate(
        [batch, jnp.full((NP - N,), NUM_GRAPHS, jnp.int32)]
    ).reshape(NP, 1)

    zer128 = jnp.zeros((128,), _f32)
    one128 = jnp.ones((128,), _f32)

    degp = _get_sc_deg()(dstf, zer128, one128)

    a2 = jnp.pad(atomic_numbers[:, None], ((0, NP - N), (0, 0)))
    pos2 = jnp.pad(pos, ((0, NP - N), (0, 0)))
    xs, dinv = _tc1(a2, pos2, degp.reshape(2, NP, 1))

    zer8 = jnp.zeros((128, 8), _f32)
    s1p = _make_edge_pass()(srcf, dstf, xs, zer8)

    z, w8, d8, bidx = _tc2(
        s1p, xs, dinv, bat_np, W1, b1.reshape(1, H), W2, Wl,
        b2.reshape(1, H), bl.reshape(1, 8)
    )

    s2p = _make_edge_pass()(srcf, dstf, z, zer8)

    poolp, cntp = _get_sc_pool()(
        s2p[0].reshape(-1), s2p[1].reshape(-1), d8.reshape(-1), w8.reshape(-1),
        bidx.reshape(-1), zer128, one128
    )

    return _tc3(poolp.reshape(2, PSLOTS // 8, 8),
                cntp.reshape(2, PSLOTS // 8, 8))


# speedup vs baseline: 1.1214x; 1.1214x over previous
import functools

import jax
import jax.numpy as jnp
from jax import lax
from jax.experimental import pallas as pl
from jax.experimental.pallas import tpu as pltpu
from jax.experimental.pallas import tpu_sc as plsc

N = 100000
E = 1600000
NUM_GRAPHS = 256
H = 32

NC, NS = 2, 16
NW = NC * NS

NP = 102400
EP = 1638400
TROWS = EP // NW // 128
GRP = 8
NGRP = TROWS // GRP
PSLOTS = 2560
BLK = 4096
NBLK = NP // BLK

_f32 = jnp.float32


@functools.cache
def _get_sc_deg():
    mesh = plsc.VectorSubcoreMesh(core_axis_name="c", subcore_axis_name="s")
    return functools.partial(
        pl.kernel,
        out_type=jax.ShapeDtypeStruct((NC, NP), _f32),
        mesh=mesh,
        compiler_params=pltpu.CompilerParams(use_tc_tiling_on_sc=False),
        scratch_types=[
            pltpu.VMEM_SHARED((NP,), _f32),
            pltpu.VMEM((128,), _f32),
            pltpu.VMEM((128,), _f32)]
            + [pltpu.VMEM((128,), jnp.int32) for _ in range(2 * GRP)]
            + [pltpu.SemaphoreType.DMA, pltpu.SemaphoreType.DMA],
    )(_sc_deg_body)


def _sc_deg_body(dstf, zer, one, degp, accd, zb, ob, *rest):
    djs = (rest[:GRP], rest[GRP:2 * GRP])
    semi, sems = rest[2 * GRP], rest[2 * GRP + 1]
    c = lax.axis_index("c")
    s = lax.axis_index("s")
    wid = c * NS + s
    pltpu.sync_copy(zer, zb)
    pltpu.sync_copy(one, ob)
    zbase = s * (NP // NS)
    zhs = [
        pltpu.async_copy(zb, accd.at[pl.ds(zbase + i * 128, 128)], semi)
        for i in range(NP // NS // 128)
    ]
    for h in zhs:
        h.wait()
    plsc.subcore_barrier()

    eb0 = wid * TROWS * 128

    def fire_idx(k, g):
        eb = eb0 + g * (GRP * 128)
        for j in range(GRP):
            pltpu.async_copy(dstf.at[pl.ds(eb + j * 128, 128)], djs[k][j], semi)

    def wait_idx(k):
        for j in range(GRP):
            pltpu.make_async_copy(dstf.at[pl.ds(0, 128)], djs[k][j], semi).wait()

    def fire_scat(k):
        for j in range(GRP):
            pltpu.async_copy(ob, accd.at[djs[k][j]], sems, add=True)

    def drain_scat(k):
        for j in range(GRP):
            pltpu.make_async_copy(ob, accd.at[djs[k][j]], sems).wait()

    def phase(k, g, drain):
        wait_idx(k)
        if drain:
            drain_scat(1 - k)
        fire_idx(1 - k, lax.rem(g + 1, NGRP))
        fire_scat(k)

    fire_idx(0, 0)
    phase(0, 0, False)

    @pl.loop(0, (NGRP - 2) // 2)
    def _g(i):
        phase(1, 1 + 2 * i, True)
        phase(0, 2 + 2 * i, True)

    phase(1, NGRP - 1, True)
    drain_scat(1)
    wait_idx(0)
    plsc.subcore_barrier()
    pltpu.sync_copy(accd.at[pl.ds(zbase, NP // NS)], degp.at[c, pl.ds(zbase, NP // NS)])


@functools.cache
def _make_edge_pass():
    D = 8
    mesh = plsc.VectorSubcoreMesh(core_axis_name="c", subcore_axis_name="s")

    @functools.partial(
        pl.kernel,
        out_type=jax.ShapeDtypeStruct((NC, NP, D), _f32),
        mesh=mesh,
        compiler_params=pltpu.CompilerParams(use_tc_tiling_on_sc=False),
        scratch_types=[
            pltpu.VMEM_SHARED((NP, D), _f32),
            pltpu.VMEM((128, D), _f32)]
            + [pltpu.VMEM((128,), jnp.int32) for _ in range(4 * GRP)]
            + [pltpu.VMEM((128, D), _f32) for _ in range(2 * GRP)]
            + [pltpu.SemaphoreType.DMA for _ in range(3)],
    )
    def _edge(srcf, dstf, tab, zer, out, acc, zb, *rest):
        sjs = (rest[:GRP], rest[GRP:2 * GRP])
        djs = (rest[2 * GRP:3 * GRP], rest[3 * GRP:4 * GRP])
        rows = (rest[4 * GRP:5 * GRP], rest[5 * GRP:6 * GRP])
        semi, semg, sems = rest[6 * GRP:6 * GRP + 3]
        c = lax.axis_index("c")
        s = lax.axis_index("s")
        wid = c * NS + s
        pltpu.sync_copy(zer, zb)
        zbase = s * (NP // NS)
        zhs = [
            pltpu.async_copy(zb, acc.at[pl.ds(zbase + i * 128, 128)], semi)
            for i in range(NP // NS // 128)
        ]
        for h in zhs:
            h.wait()
        plsc.subcore_barrier()

        eb0 = wid * TROWS * 128

        def fire_idx(k, g):
            eb = eb0 + g * (GRP * 128)
            for j in range(GRP):
                pltpu.async_copy(srcf.at[pl.ds(eb + j * 128, 128)], sjs[k][j], semi)
                pltpu.async_copy(dstf.at[pl.ds(eb + j * 128, 128)], djs[k][j], semi)

        def wait_idx(k):
            for j in range(GRP):
                pltpu.make_async_copy(srcf.at[pl.ds(0, 128)], sjs[k][j], semi).wait()
                pltpu.make_async_copy(dstf.at[pl.ds(0, 128)], djs[k][j], semi).wait()

        def drain_scat(k):
            for j in range(GRP):
                pltpu.make_async_copy(rows[k][j], acc.at[djs[k][j]], sems).wait()

        def phase(k, g, drain):
            wait_idx(k)
            ghs = [
                pltpu.async_copy(tab.at[sjs[k][j]], rows[k][j], semg)
                for j in range(GRP)
            ]
            if drain:
                drain_scat(1 - k)
            fire_idx(1 - k, lax.rem(g + 1, NGRP))
            for h in ghs:
                h.wait()
            for j in range(GRP):
                pltpu.async_copy(rows[k][j], acc.at[djs[k][j]], sems, add=True)

        fire_idx(0, 0)
        phase(0, 0, False)

        @pl.loop(0, (NGRP - 2) // 2)
        def _g(i):
            phase(1, 1 + 2 * i, True)
            phase(0, 2 + 2 * i, True)

        phase(1, NGRP - 1, True)
        drain_scat(1)
        wait_idx(0)
        plsc.subcore_barrier()
        pltpu.sync_copy(
            acc.at[pl.ds(zbase, NP // NS)], out.at[c, pl.ds(zbase, NP // NS)]
        )

    return _edge


@functools.cache
def _get_sc_pool():
    mesh = plsc.VectorSubcoreMesh(core_axis_name="c", subcore_axis_name="s")
    return functools.partial(
        pl.kernel,
        out_type=(
            jax.ShapeDtypeStruct((NC, PSLOTS), _f32),
            jax.ShapeDtypeStruct((NC, PSLOTS), _f32),
        ),
        mesh=mesh,
        compiler_params=pltpu.CompilerParams(use_tc_tiling_on_sc=False),
        scratch_types=[
            pltpu.VMEM_SHARED((PSLOTS,), _f32),
            pltpu.VMEM_SHARED((PSLOTS,), _f32),
            pltpu.VMEM((128,), _f32),
            pltpu.VMEM((128,), _f32),
            pltpu.VMEM((1024,), _f32),
            pltpu.VMEM((1024,), _f32),
            pltpu.VMEM((1024,), _f32),
            pltpu.VMEM((1024,), _f32)]
            + [pltpu.VMEM((128,), _f32) for _ in range(GRP)]
            + [pltpu.VMEM((128,), jnp.int32) for _ in range(GRP)]
            + [pltpu.SemaphoreType.DMA, pltpu.SemaphoreType.DMA],
    )(_sc_pool_body)


def _sc_pool_body(s2a, s2b, d8, w8, bidxf, zer, one, poolp, cntp,
                  accp, accq, zb, ob, ab, bb, db, wb, *rest):
    lbs = rest[:GRP]
    ibs = rest[GRP:2 * GRP]
    semi, sems = rest[2 * GRP], rest[2 * GRP + 1]
    c = lax.axis_index("c")
    s = lax.axis_index("s")
    wid = c * NS + s
    pltpu.sync_copy(zer, zb)
    pltpu.sync_copy(one, ob)

    @pl.when(s == 0)
    def _zp():
        for k in range(PSLOTS // 128):
            pltpu.sync_copy(zb, accp.at[pl.ds(k * 128, 128)])

    @pl.when(s == 1)
    def _zq():
        for k in range(PSLOTS // 128):
            pltpu.sync_copy(zb, accq.at[pl.ds(k * 128, 128)])

    plsc.subcore_barrier()

    eb0 = wid * (NP * 8 // NW)

    def fire_data(g):
        eb = eb0 + g * 1024
        pltpu.async_copy(s2a.at[pl.ds(eb, 1024)], ab, semi)
        pltpu.async_copy(s2b.at[pl.ds(eb, 1024)], bb, semi)
        pltpu.async_copy(d8.at[pl.ds(eb, 1024)], db, semi)
        pltpu.async_copy(w8.at[pl.ds(eb, 1024)], wb, semi)

    def fire_idx(g):
        eb = eb0 + g * 1024
        for j in range(GRP):
            pltpu.async_copy(bidxf.at[pl.ds(eb + j * 128, 128)], ibs[j], semi)

    def wait_loads():
        for r in (ab, bb, db, wb):
            pltpu.make_async_copy(s2a.at[pl.ds(0, 1024)], r, semi).wait()
        for j in range(GRP):
            pltpu.make_async_copy(bidxf.at[pl.ds(0, 128)], ibs[j], semi).wait()

    def drain_scat():
        for j in range(GRP):
            pltpu.make_async_copy(lbs[j], accp.at[ibs[j]], sems).wait()
            pltpu.make_async_copy(ob, accq.at[ibs[j]], sems).wait()

    def body(g, drain):
        fire_data(g)
        if drain:
            drain_scat()
        fire_idx(g)
        wait_loads()
        for j in range(GRP):
            for k in range(8):
                sl = pl.ds(j * 128 + k * 16, 16)
                lbs[j][pl.ds(k * 16, 16)] = db[sl] * (ab[sl] + bb[sl]) + wb[sl]
        for j in range(GRP):
            pltpu.async_copy(lbs[j], accp.at[ibs[j]], sems, add=True)
            pltpu.async_copy(ob, accq.at[ibs[j]], sems, add=True)

    body(0, False)

    @pl.loop(1, NP * 8 // NW // 1024)
    def _p(i):
        body(i, True)

    drain_scat()
    plsc.subcore_barrier()

    @pl.when(s == 0)
    def _op():
        pltpu.sync_copy(accp, poolp.at[c])

    @pl.when(s == 1)
    def _oq():
        pltpu.sync_copy(accq, cntp.at[c])


def _tc1_body(a_ref, p_ref, degp_ref, xs_ref, dinv_ref):
    i = pl.program_id(0)
    deg = degp_ref[0] + degp_ref[1] + 1.0
    dinv = lax.rsqrt(deg)
    x = jnp.concatenate(
        [a_ref[...], p_ref[...], jnp.zeros((BLK, 4), _f32)], axis=-1
    )
    rid = lax.broadcasted_iota(jnp.int32, (BLK, 1), 0) + i * BLK
    xs_ref[...] = jnp.where(rid < N, x * dinv, 0.0)
    dinv_ref[...] = dinv


def _tc1(a2, pos2, degp3):
    return pl.pallas_call(
        _tc1_body,
        grid=(NBLK,),
        in_specs=[
            pl.BlockSpec((BLK, 1), lambda i: (i, 0)),
            pl.BlockSpec((BLK, 3), lambda i: (i, 0)),
            pl.BlockSpec((2, BLK, 1), lambda i: (0, i, 0)),
        ],
        out_specs=[
            pl.BlockSpec((BLK, 8), lambda i: (i, 0)),
            pl.BlockSpec((BLK, 1), lambda i: (i, 0)),
        ],
        out_shape=[
            jax.ShapeDtypeStruct((NP, 8), _f32),
            jax.ShapeDtypeStruct((NP, 1), _f32),
        ],
    )(a2, pos2, degp3)


def _tc2_body(s1_ref, xs_ref, dinv_ref, bat_ref, W1_ref, b1_ref, W2_ref,
              Wl_ref, b2_ref, bl_ref, z_ref, w_ref, d8_ref, bidx_ref):
    dinv = dinv_ref[...]
    agg1 = (s1_ref[0, :, :4] + s1_ref[1, :, :4] + xs_ref[:, :4]) * dinv
    W1 = W1_ref[...]
    x1 = b1_ref[...]
    for k in range(4):
        x1 = x1 + agg1[:, k:k + 1] * W1[k:k + 1, :]
    y = jnp.maximum(x1, 0.0)
    M = jnp.dot(W2_ref[...], Wl_ref[...], preferred_element_type=_f32)
    z = jnp.dot(y * dinv, M, preferred_element_type=_f32)
    cvec = jnp.dot(b2_ref[...], Wl_ref[...], preferred_element_type=_f32) + bl_ref[...]
    z_ref[...] = z
    w_ref[...] = dinv * z + cvec
    d8_ref[...] = jnp.broadcast_to(dinv, (BLK, 8))
    bidx_ref[...] = bat_ref[...] * 8 + lax.broadcasted_iota(jnp.int32, (BLK, 8), 1)


def _tc2(s1p, xs, dinv, bat2, W1, b1, W2, Wl, b2, bl):
    wspec = lambda r, c_: pl.BlockSpec((r, c_), lambda i: (0, 0))
    return pl.pallas_call(
        _tc2_body,
        grid=(NBLK,),
        in_specs=[
            pl.BlockSpec((2, BLK, 8), lambda i: (0, i, 0)),
            pl.BlockSpec((BLK, 8), lambda i: (i, 0)),
            pl.BlockSpec((BLK, 1), lambda i: (i, 0)),
            pl.BlockSpec((BLK, 1), lambda i: (i, 0)),
            wspec(4, 32), wspec(1, 32), wspec(32, 32),
            wspec(32, 8), wspec(1, 32), wspec(1, 8),
        ],
        out_specs=[
            pl.BlockSpec((BLK, 8), lambda i: (i, 0)),
            pl.BlockSpec((BLK, 8), lambda i: (i, 0)),
            pl.BlockSpec((BLK, 8), lambda i: (i, 0)),
            pl.BlockSpec((BLK, 8), lambda i: (i, 0)),
        ],
        out_shape=[
            jax.ShapeDtypeStruct((NP, 8), _f32),
            jax.ShapeDtypeStruct((NP, 8), _f32),
            jax.ShapeDtypeStruct((NP, 8), _f32),
            jax.ShapeDtypeStruct((NP, 8), jnp.int32),
        ],
    )(s1p, xs, dinv, bat2, W1, b1, W2, Wl, b2, bl)


def _tc3_body(pool_ref, cnt_ref, out_ref):
    sums = pool_ref[0] + pool_ref[1]
    cnt = (cnt_ref[0] + cnt_ref[1])[:, 0:1]
    pooled = sums[:NUM_GRAPHS] / jnp.maximum(cnt[:NUM_GRAPHS], 1.0)
    m = jnp.max(pooled, axis=1, keepdims=True)
    lse = jnp.log(jnp.sum(jnp.exp(pooled - m), axis=1, keepdims=True)) + m
    out_ref[...] = pooled - lse


def _tc3(poolp3, cntp3):
    return pl.pallas_call(
        _tc3_body,
        grid=(1,),
        in_specs=[
            pl.BlockSpec((2, PSLOTS // 8, 8), lambda i: (0, 0, 0)),
            pl.BlockSpec((2, PSLOTS // 8, 8), lambda i: (0, 0, 0)),
        ],
        out_specs=pl.BlockSpec((NUM_GRAPHS, 8), lambda i: (0, 0)),
        out_shape=jax.ShapeDtypeStruct((NUM_GRAPHS, 8), _f32),
    )(poolp3, cntp3)


def kernel(atomic_numbers, pos, edge_index, batch, W1, b1, W2, b2, Wl, bl):
    src = edge_index[0]
    dst = edge_index[1]
    pad_e = EP - E
    srcf = jnp.concatenate([src, jnp.zeros((pad_e,), jnp.int32)])
    dstf = jnp.concatenate([dst, jnp.full((pad_e,), N, jnp.int32)])
    bat_np = jnp.concatenate(
        [batch, jnp.full((NP - N,), NUM_GRAPHS, jnp.int32)]
    ).reshape(NP, 1)

    zer128 = jnp.zeros((128,), _f32)
    one128 = jnp.ones((128,), _f32)

    degp = _get_sc_deg()(dstf, zer128, one128)

    a2 = jnp.pad(atomic_numbers[:, None], ((0, NP - N), (0, 0)))
    pos2 = jnp.pad(pos, ((0, NP - N), (0, 0)))
    xs, dinv = _tc1(a2, pos2, degp.reshape(2, NP, 1))

    zer8 = jnp.zeros((128, 8), _f32)
    s1p = _make_edge_pass()(srcf, dstf, xs, zer8)

    z, w8, d8, bidx = _tc2(
        s1p, xs, dinv, bat_np, W1, b1.reshape(1, H), W2, Wl,
        b2.reshape(1, H), bl.reshape(1, 8)
    )

    s2p = _make_edge_pass()(srcf, dstf, z, zer8)

    poolp, cntp = _get_sc_pool()(
        s2p[0].reshape(-1), s2p[1].reshape(-1), d8.reshape(-1), w8.reshape(-1),
        bidx.reshape(-1), zer128, one128
    )

    return _tc3(poolp.reshape(2, PSLOTS // 8, 8),
                cntp.reshape(2, PSLOTS // 8, 8))
